# baseline (speedup 1.0000x reference)
import functools

import jax
import jax.numpy as jnp
from jax import lax
from jax.experimental import pallas as pl
from jax.experimental.pallas import tpu as pltpu
from jax.experimental.pallas import tpu_sc as plsc

N = 10000
NP = 10240
E = 320000
D1 = 128
D2 = 64
NC = 2
NS = 16
NW = NC * NS
CH = 80
ROWS_IDX = E // CH
RPT = NP // NS
NG = 5
GR = ROWS_IDX // NW // NG

_mesh = plsc.VectorSubcoreMesh(core_axis_name="c", subcore_axis_name="s")
_sc_params = pltpu.CompilerParams(use_tc_tiling_on_sc=False)


@functools.partial(
    pl.kernel,
    out_type=jax.ShapeDtypeStruct((NC, NP), jnp.float32),
    mesh=_mesh,
    scratch_types=[
        pltpu.VMEM((ROWS_IDX // NW, CH), jnp.int32),
        pltpu.VMEM((CH,), jnp.float32),
        pltpu.VMEM((RPT,), jnp.float32),
        pltpu.VMEM_SHARED((NP,), jnp.float32),
        pltpu.SemaphoreType.DMA,
        pltpu.SemaphoreType.DMA,
        pltpu.SemaphoreType.DMA,
        pltpu.SemaphoreType.DMA,
        pltpu.SemaphoreType.DMA,
    ],
    compiler_params=_sc_params,
)
def _deg_sc(dst_hbm, deg_hbm, idx_v, ones_v, zb_v, acc_sh,
            ds0, ds1, ds2, ds3, ds4):
    c = lax.axis_index("c")
    s = lax.axis_index("s")
    nrows = ROWS_IDX // NW

    o16 = jnp.ones((16,), jnp.float32)
    z16 = jnp.zeros((16,), jnp.float32)
    for i in range(CH // 16):
        ones_v[pl.ds(i * 16, 16)] = o16

    def zb(j, carry):
        zb_v[pl.ds(j * 16, 16)] = z16
        return carry

    lax.fori_loop(0, RPT // 16, zb, 0)
    pltpu.sync_copy(zb_v, acc_sh.at[pl.ds(s * RPT, RPT)])
    pltpu.sync_copy(dst_hbm.at[c * NS + s], idx_v)

    plsc.subcore_barrier()

    dsem = [ds0, ds1, ds2, ds3, ds4]
    ngrp = nrows // 25

    def step(g, carry):
        sd = {}
        for j in range(25):
            b = j % 5
            if j >= 5:
                sd.pop(j - 5).wait()
            row = g * 25 + j
            sd[j] = pltpu.async_copy(ones_v, acc_sh.at[idx_v.at[row]],
                                     dsem[b], add=True)
        for j in range(20, 25):
            sd.pop(j).wait()
        return carry

    lax.fori_loop(0, ngrp, step, 0)

    plsc.subcore_barrier()
    pltpu.sync_copy(acc_sh.at[pl.ds(s * RPT, RPT)],
                    deg_hbm.at[c, pl.ds(s * RPT, RPT)])


NRT = N // NS


def _make_agg_flat(D, nslot):
    TOT = ROWS_IDX // NW

    @functools.partial(
        pl.kernel,
        out_type=jax.ShapeDtypeStruct((NC, N, D), jnp.float32),
        mesh=_mesh,
        scratch_types=(
            [
                pltpu.VMEM((2, GR, CH), jnp.int32),
                pltpu.VMEM((2, GR, CH), jnp.int32),
                pltpu.VMEM((nslot, CH, D), jnp.float32),
                pltpu.VMEM_SHARED((N, D), jnp.float32),
            ]
            + [pltpu.SemaphoreType.DMA] * (2 * nslot + 2)
        ),
        compiler_params=_sc_params,
    )
    def agg(hs_hbm, src_hbm, dst_hbm, out_hbm, si_v, di_v, rows_v,
            acc_sh, *sems):
        c = lax.axis_index("c")
        s = lax.axis_index("s")
        z16 = jnp.zeros((16,), jnp.float32)
        gsem = list(sems[:nslot])
        ssem = list(sems[nslot:2 * nslot])
        isem = list(sems[2 * nslot:])
        wid = c * NS + s

        idxd = {0: [pltpu.async_copy(src_hbm.at[wid, 0], si_v.at[0], isem[0]),
                    pltpu.async_copy(dst_hbm.at[wid, 0], di_v.at[0], isem[0])]}

        def zb(j, carry):
            for i in range(D // 16):
                rows_v[0, j, pl.ds(i * 16, 16)] = z16
            return carry

        lax.fori_loop(0, CH, zb, 0)
        zd = []
        for t in range(NRT // CH):
            zd.append(pltpu.async_copy(
                rows_v.at[0], acc_sh.at[pl.ds(s * NRT + t * CH, CH)],
                ssem[t % nslot]))
        rem = NRT % CH
        if rem:
            zd.append(pltpu.async_copy(
                rows_v.at[0, pl.ds(0, rem)],
                acc_sh.at[pl.ds(s * NRT + NRT - rem, rem)],
                ssem[(NRT // CH) % nslot]))
        for d in zd:
            d.wait()
        for d in idxd[0]:
            d.wait()
        idx_ready = {0}

        def fire(k):
            g = k // GR
            return pltpu.async_copy(
                hs_hbm.at[si_v.at[g % 2, k - g * GR]],
                rows_v.at[k % nslot], gsem[k % nslot])

        gd = {k: fire(k) for k in range(nslot - 1)}

        plsc.subcore_barrier()

        sd = {}
        for jj in range(TOT):
            g = jj // GR
            b = jj % nslot
            fk = jj + nslot - 1
            if fk < TOT:
                fg = fk // GR
                if fg not in idx_ready:
                    for d in idxd.pop(fg):
                        d.wait()
                    idx_ready.add(fg)
                if jj - 1 >= 0:
                    sd.pop(jj - 1).wait()
                gd[fk] = fire(fk)
            gd.pop(jj).wait()
            sd[jj] = pltpu.async_copy(
                rows_v.at[b], acc_sh.at[di_v.at[g % 2, jj - g * GR]],
                ssem[b], add=True)
            if jj % GR == 0 and g + 1 < NG:
                pb = (g + 1) % 2
                idxd[g + 1] = [
                    pltpu.async_copy(src_hbm.at[wid, g + 1], si_v.at[pb],
                                     isem[pb]),
                    pltpu.async_copy(dst_hbm.at[wid, g + 1], di_v.at[pb],
                                     isem[pb]),
                ]
        for jj in range(max(0, TOT - nslot), TOT):
            if jj in sd:
                sd.pop(jj).wait()

        plsc.subcore_barrier()
        pltpu.sync_copy(acc_sh.at[pl.ds(s * NRT, NRT)],
                        out_hbm.at[c, pl.ds(s * NRT, NRT)])

    return agg


_agg1 = _make_agg_flat(D1, 3)
_agg2 = _make_agg_flat(D2, 6)


RB = 1000


def _tc1(x, W1, degc):
    def body(x_ref, w_ref, d0_ref, d1_ref, o_ref):
        h = jnp.dot(x_ref[...], w_ref[...], preferred_element_type=jnp.float32)
        dinv = lax.rsqrt(d0_ref[...] + d1_ref[...] + 1.0)
        o_ref[...] = h * dinv

    nb = N // RB
    return pl.pallas_call(
        body,
        grid=(nb,),
        in_specs=[
            pl.BlockSpec((RB, D1), lambda i: (i, 0)),
            pl.BlockSpec((D1, D1), lambda i: (0, 0)),
            pl.BlockSpec((RB, 1), lambda i: (i, 0)),
            pl.BlockSpec((RB, 1), lambda i: (nb + i, 0)),
        ],
        out_specs=pl.BlockSpec((RB, D1), lambda i: (i, 0)),
        out_shape=jax.ShapeDtypeStruct((N, D1), jnp.float32),
    )(x, W1, degc, degc)


def _tc2(agg1, hs1, degc, W2, b1r):
    def body(a_ref, h_ref, d0_ref, d1_ref, w_ref, b_ref, o_ref):
        dinv = lax.rsqrt(d0_ref[...] + d1_ref[...] + 1.0)
        pre = (a_ref[0] + a_ref[1] + h_ref[...]) * dinv + b_ref[...]
        act = jnp.maximum(pre, 0.0)
        h2 = jnp.dot(act, w_ref[...], preferred_element_type=jnp.float32)
        o_ref[...] = h2 * dinv

    nb = N // RB
    return pl.pallas_call(
        body,
        grid=(nb,),
        in_specs=[
            pl.BlockSpec((NC, RB, D1), lambda i: (0, i, 0)),
            pl.BlockSpec((RB, D1), lambda i: (i, 0)),
            pl.BlockSpec((RB, 1), lambda i: (i, 0)),
            pl.BlockSpec((RB, 1), lambda i: (nb + i, 0)),
            pl.BlockSpec((D1, D2), lambda i: (0, 0)),
            pl.BlockSpec((1, D1), lambda i: (0, 0)),
        ],
        out_specs=pl.BlockSpec((RB, D2), lambda i: (i, 0)),
        out_shape=jax.ShapeDtypeStruct((N, D2), jnp.float32),
    )(agg1, hs1, degc, degc, W2, b1r)


def _tc3(agg2, hs2, degc, b2r):
    def body(a_ref, h_ref, d0_ref, d1_ref, b_ref, o_ref):
        dinv = lax.rsqrt(d0_ref[...] + d1_ref[...] + 1.0)
        o = (a_ref[0] + a_ref[1] + h_ref[...]) * dinv + b_ref[...]
        m = jnp.max(o, axis=1, keepdims=True)
        e = jnp.exp(o - m)
        lse = jnp.log(jnp.sum(e, axis=1, keepdims=True))
        o_ref[...] = o - m - lse

    nb = N // RB
    return pl.pallas_call(
        body,
        grid=(nb,),
        in_specs=[
            pl.BlockSpec((NC, RB, D2), lambda i: (0, i, 0)),
            pl.BlockSpec((RB, D2), lambda i: (i, 0)),
            pl.BlockSpec((RB, 1), lambda i: (i, 0)),
            pl.BlockSpec((RB, 1), lambda i: (nb + i, 0)),
            pl.BlockSpec((1, D2), lambda i: (0, 0)),
        ],
        out_specs=pl.BlockSpec((RB, D2), lambda i: (i, 0)),
        out_shape=jax.ShapeDtypeStruct((N, D2), jnp.float32),
    )(agg2, hs2, degc, degc, b2r)


def kernel(x, edge_index, eigenvectors, W1, b1, W2, b2):
    del eigenvectors
    src4 = edge_index[0].reshape(NW, NG, GR, CH)
    dst4 = edge_index[1].reshape(NW, NG, GR, CH)
    dst3 = edge_index[1].reshape(NW, ROWS_IDX // NW, CH)

    deg = _deg_sc(dst3)
    degc = deg[:, :N].reshape(NC * N, 1)

    hs1 = _tc1(x, W1, degc)
    agg1 = _agg1(hs1, src4, dst4)
    hs2 = _tc2(agg1, hs1, degc, W2, b1.reshape(1, D1))
    agg2 = _agg2(hs2, src4, dst4)
    return _tc3(agg2, hs2, degc, b2.reshape(1, D2))
